# baseline (speedup 1.0000x reference)
import functools

import jax
import jax.numpy as jnp
from jax import lax
from jax.experimental import pallas as pl
from jax.experimental.pallas import tpu as pltpu
from jax.experimental.pallas import tpu_sc as plsc

HOURS = 168
SU, SL, TU, TL = 100.0, 0.0, 3600.0, 0.0
EMB = 32
N, M = 1024, 20
NUM_LOC = 1000000
NUM_USER = 100000

NC, NS = 2, 16
NW = NC * NS
ROWS = N * M
BPW = ROWS // NW

BR = 2560
LANES = 128


def _gather_body(idx_t, idx_l, idx_u, tab_t, tab_l, tab_u,
                 out_t, out_l, out_u, idx_v, row_v, sem):
    wid = lax.axis_index("s") * NC + lax.axis_index("c")
    base = wid * BPW
    for idx_hbm, tab_hbm, out_hbm in ((idx_t, tab_t, out_t),
                                      (idx_l, tab_l, out_l),
                                      (idx_u, tab_u, out_u)):
        pltpu.sync_copy(idx_hbm.at[pl.ds(base, BPW)], idx_v)
        pltpu.async_copy(tab_hbm.at[idx_v], row_v, sem).wait()
        pltpu.sync_copy(row_v, out_hbm.at[pl.ds(base, BPW)])


@functools.lru_cache(maxsize=1)
def _gather_call():
    return pl.kernel(
        _gather_body,
        out_type=(jax.ShapeDtypeStruct((ROWS, 4 * EMB), jnp.float32),
                  jax.ShapeDtypeStruct((ROWS, 4 * EMB), jnp.float32),
                  jax.ShapeDtypeStruct((ROWS, 4 * EMB), jnp.float32)),
        mesh=plsc.VectorSubcoreMesh(core_axis_name="c", subcore_axis_name="s",
                                    num_cores=NC, num_subcores=NS),
        scratch_types=[
            pltpu.VMEM((BPW,), jnp.int32),
            pltpu.VMEM((BPW, 4 * EMB), jnp.float32),
            pltpu.SemaphoreType.DMA,
        ],
    )


def _pick(row_ref, off_ref):
    offv = off_ref[...]
    picked = row_ref[:, 3 * EMB:4 * EMB]
    for g in (2, 1, 0):
        picked = jnp.where(offv == g, row_ref[:, g * EMB:(g + 1) * EMB], picked)
    return picked


def _tile20(row):
    return jnp.concatenate([row] * M, axis=1)


def _fused_body(len_ref, mat_ref, esl, etl, w_ref,
                rt, rl, ru, ot, ol, ou, joint_ref, delta_ref):
    joint_ref[...] = _pick(rt, ot) + _pick(rl, ol) + _pick(ru, ou)

    lenv = len_ref[...]
    i_row = lax.broadcasted_iota(jnp.int32, (BR, 1), 0) % M
    vi = i_row < lenv

    matv = mat_ref[...]
    j2 = lax.broadcasted_iota(jnp.int32, (1, 2 * M), 1) // 2
    m40 = (vi & (j2 < lenv)).astype(jnp.float32)
    x = jnp.concatenate([matv, matv * m40,
                         jnp.zeros((BR, 128 - 4 * M), jnp.float32)], axis=1)
    y = jax.lax.dot_general(x.astype(jnp.bfloat16), w_ref[...],
                            (((1,), (0,)), ((), ())),
                            preferred_element_type=jnp.float32)

    A = esl[...] + etl[...]
    A0, A1 = _tile20(A[0:1, :]), _tile20(A[1:2, :])
    j640 = lax.broadcasted_iota(jnp.int32, (1, M * EMB), 1) // EMB
    m640 = vi & (j640 < lenv)
    delta_ref[...] = y + jnp.where(m640, A1, A0)


_row_spec = pl.BlockSpec((BR, 4 * EMB), lambda i: (i, 0))
_off_spec = pl.BlockSpec((BR, 1), lambda i: (i, 0))
_tab_spec = pl.BlockSpec((2, EMB), lambda i: (0, 0))

_fused_call = pl.pallas_call(
    _fused_body,
    grid=(ROWS // BR,),
    in_specs=[
        _off_spec,
        pl.BlockSpec((BR, 2 * M), lambda i: (i, 0)),
        _tab_spec, _tab_spec,
        pl.BlockSpec((128, M * EMB), lambda i: (0, 0)),
        _row_spec, _row_spec, _row_spec,
        _off_spec, _off_spec, _off_spec,
    ],
    out_specs=[
        pl.BlockSpec((BR, EMB), lambda i: (i, 0)),
        pl.BlockSpec((BR, M * EMB), lambda i: (i, 0)),
    ],
    out_shape=[
        jax.ShapeDtypeStruct((ROWS, EMB), jnp.float32),
        jax.ShapeDtypeStruct((ROWS, M * EMB), jnp.float32),
    ],
)


def kernel(traj, mat, traj_len, emb_t, emb_l, emb_u, emb_su, emb_sl, emb_tu, emb_tl):
    traj = traj.astype(jnp.int32)
    idx_u = traj[:, :, 0].reshape(ROWS)
    idx_l = traj[:, :, 1].reshape(ROWS)
    idx_t = ((traj[:, :, 2] - 1) % HOURS + 1).reshape(ROWS)

    tab_t = jnp.pad(emb_t, ((0, 3), (0, 0))).reshape((HOURS + 4) // 4, 4 * EMB)
    tab_l = emb_l.reshape(NUM_LOC // 4, 4 * EMB)
    tab_u = emb_u.reshape(NUM_USER // 4, 4 * EMB)

    rt, rl, ru = _gather_call()(idx_t // 4, idx_l // 4, idx_u // 4,
                                tab_t, tab_l, tab_u)

    lenr = jnp.repeat(traj_len.astype(jnp.int32), M).reshape(ROWS, 1)
    mat2 = mat.reshape(ROWS, 2 * M)

    S0 = (emb_su[0] - emb_sl[0]) * (1.0 / (SU - SL))
    T0 = (emb_tu[0] - emb_tl[0]) * (1.0 / (TU - TL))
    dS = (emb_su[1] - emb_sl[1]) * (1.0 / (SU - SL)) - S0
    dT = (emb_tu[1] - emb_tl[1]) * (1.0 / (TU - TL)) - T0
    sel = (jnp.arange(M * EMB) // EMB == jnp.arange(M)[:, None]).astype(jnp.float32)
    def rows(vs, vt):
        ws = sel * jnp.tile(vs, M)[None, :]
        wt = sel * jnp.tile(vt, M)[None, :]
        return jnp.stack([ws, wt], axis=1).reshape(2 * M, M * EMB)
    w = jnp.concatenate([rows(S0, T0), rows(dS, dT),
                         jnp.zeros((128 - 4 * M, M * EMB), jnp.float32)],
                        axis=0).astype(jnp.bfloat16)

    joint, delta = _fused_call(
        lenr, mat2, emb_sl, emb_tl, w,
        rt, rl, ru,
        (idx_t % 4).reshape(ROWS, 1),
        (idx_l % 4).reshape(ROWS, 1),
        (idx_u % 4).reshape(ROWS, 1),
    )
    return joint.reshape(N, M, EMB), delta.reshape(N, M, M, EMB)


# speedup vs baseline: 2.0435x; 2.0435x over previous
import functools

import jax
import jax.numpy as jnp
from jax import lax
from jax.experimental import pallas as pl
from jax.experimental.pallas import tpu as pltpu
from jax.experimental.pallas import tpu_sc as plsc

HOURS = 168
SU, SL, TU, TL = 100.0, 0.0, 3600.0, 0.0
EMB = 32
N, M = 1024, 20
NUM_LOC = 1000000
NUM_USER = 100000

NC, NS = 2, 16
NW = NC * NS
ROWS = N * M
BPW = ROWS // NW

BR = 2560
BN = 128
Q_L = 256000
Q_U = 25600


def _gather_body(idx_t, idx_l, idx_u, tab_t, tab_l, tab_u,
                 out_t, out_l, out_u, idx_v, row_v, sem):
    wid = lax.axis_index("s") * NC + lax.axis_index("c")
    base = wid * BPW
    for idx_hbm, tab_hbm, out_hbm in ((idx_t, tab_t, out_t),
                                      (idx_l, tab_l, out_l),
                                      (idx_u, tab_u, out_u)):
        pltpu.sync_copy(idx_hbm.at[pl.ds(base, BPW)], idx_v)
        pltpu.async_copy(tab_hbm.at[idx_v], row_v, sem).wait()
        pltpu.sync_copy(row_v, out_hbm.at[pl.ds(base, BPW)])


@functools.lru_cache(maxsize=1)
def _gather_call():
    return pl.kernel(
        _gather_body,
        out_type=(jax.ShapeDtypeStruct((ROWS, 4 * EMB), jnp.float32),
                  jax.ShapeDtypeStruct((ROWS, 4 * EMB), jnp.float32),
                  jax.ShapeDtypeStruct((ROWS, 4 * EMB), jnp.float32)),
        mesh=plsc.VectorSubcoreMesh(core_axis_name="c", subcore_axis_name="s",
                                    num_cores=NC, num_subcores=NS),
        scratch_types=[
            pltpu.VMEM((BPW,), jnp.int32),
            pltpu.VMEM((BPW, 4 * EMB), jnp.float32),
            pltpu.SemaphoreType.DMA,
        ],
        compiler_params=pltpu.CompilerParams(use_tc_tiling_on_sc=False),
    )


def _pack_body(in_hbm, out_hbm, buf, obuf, buf_t, sem_in, sem_out,
               *, nv, nv4, pb):
    steps = nv4 // pb
    step = pl.program_id(0)
    slot = lax.rem(step, 2)
    l3 = nv - 3 * nv4
    s_part, w_t = divmod(l3, pb)

    def copy_q(s, k, slot_):
        return pltpu.make_async_copy(
            in_hbm.at[:, pl.ds(k * nv4 + s * pb, pb)],
            buf.at[slot_, k], sem_in.at[slot_])

    def copy_t(slot_):
        return pltpu.make_async_copy(
            in_hbm.at[:, pl.ds(3 * nv4 + s_part * pb, w_t)],
            buf_t, sem_in.at[slot_])

    def start_in(s, slot_):
        for k in range(3):
            copy_q(s, k, slot_).start()

        @pl.when((s + 1) * pb <= l3)
        def _():
            copy_q(s, 3, slot_).start()

        if w_t:
            @pl.when(s == s_part)
            def _():
                copy_t(slot_).start()

    def out_dma(s, slot_):
        return pltpu.make_async_copy(obuf.at[slot_],
                                     out_hbm.at[pl.ds(s * pb, pb)],
                                     sem_out.at[slot_])

    @pl.when(step == 0)
    def _():
        start_in(step, slot)

    @pl.when(step + 1 < steps)
    def _():
        start_in(step + 1, 1 - slot)

    @pl.when(step >= 2)
    def _():
        out_dma(step - 2, slot).wait()

    for k in range(3):
        copy_q(step, k, slot).wait()
        obuf[slot, :, pl.ds(k * EMB, EMB)] = jnp.transpose(buf[slot, k])

    @pl.when((step + 1) * pb <= l3)
    def _():
        copy_q(step, 3, slot).wait()
        obuf[slot, :, pl.ds(3 * EMB, EMB)] = jnp.transpose(buf[slot, 3])

    if w_t:
        @pl.when(step == s_part)
        def _():
            copy_t(slot).wait()
            obuf[slot, pl.ds(0, w_t), pl.ds(3 * EMB, EMB)] = (
                jnp.transpose(buf_t[...]))

    out_dma(step, slot).start()

    @pl.when(step == steps - 1)
    def _():
        out_dma(step, slot).wait()
        out_dma(step - 1, 1 - slot).wait()


def _pack_call(nv, nv4, pb):
    w_t = (nv - 3 * nv4) % pb
    return pl.pallas_call(
        functools.partial(_pack_body, nv=nv, nv4=nv4, pb=pb),
        grid=(nv4 // pb,),
        in_specs=[pl.BlockSpec(memory_space=pl.ANY)],
        out_specs=pl.BlockSpec(memory_space=pl.ANY),
        out_shape=jax.ShapeDtypeStruct((nv4, 4 * EMB), jnp.float32),
        scratch_shapes=[
            pltpu.VMEM((2, 4, EMB, pb), jnp.float32),
            pltpu.VMEM((2, pb, 4 * EMB), jnp.float32),
            pltpu.VMEM((EMB, w_t if w_t else 128), jnp.float32),
            pltpu.SemaphoreType.DMA((2,)),
            pltpu.SemaphoreType.DMA((2,)),
        ],
        compiler_params=pltpu.CompilerParams(
            dimension_semantics=("arbitrary",)),
    )


def _pick(row_ref, off_ref):
    offv = off_ref[...]
    picked = row_ref[:, 3 * EMB:4 * EMB]
    for g in (2, 1, 0):
        picked = jnp.where(offv == g, row_ref[:, g * EMB:(g + 1) * EMB], picked)
    return picked


def _joint_body(rt, rl, ru, ot, ol, ou, joint_ref):
    joint_ref[...] = _pick(rt, ot) + _pick(rl, ol) + _pick(ru, ou)


_row_spec = pl.BlockSpec((BR, 4 * EMB), lambda i: (i, 0))
_off_spec = pl.BlockSpec((BR, 1), lambda i: (i, 0))

_joint_call = pl.pallas_call(
    _joint_body,
    grid=(ROWS // BR,),
    in_specs=[_row_spec, _row_spec, _row_spec, _off_spec, _off_spec, _off_spec],
    out_specs=pl.BlockSpec((BR, EMB), lambda i: (i, 0)),
    out_shape=jax.ShapeDtypeStruct((ROWS, EMB), jnp.float32),
)


def _delta_body(len_ref, matT_ref, cst_ref, delta_ref):
    lenn = len_ref[...]
    a0, a1 = cst_ref[:, 0:1], cst_ref[:, 1:2]
    s0, s1 = cst_ref[:, 2:3], cst_ref[:, 3:4]
    t0, t1 = cst_ref[:, 4:5], cst_ref[:, 5:6]
    vjs = [j < lenn for j in range(M)]
    for i in range(M):
        vi = vjs[i]
        for j in range(M):
            m = vi & vjs[j]
            ds = matT_ref[i, j, 0:1, :]
            dt = matT_ref[i, j, 1:2, :]
            am = jnp.where(m, a1, a0)
            sm = jnp.where(m, s1, s0)
            tm = jnp.where(m, t1, t0)
            delta_ref[i, j] = am + ds * sm + dt * tm


_delta_call = pl.pallas_call(
    _delta_body,
    grid=(N // BN,),
    in_specs=[
        pl.BlockSpec((1, BN), lambda i: (0, i)),
        pl.BlockSpec((M, M, 2, BN), lambda i: (0, 0, 0, i)),
        pl.BlockSpec((EMB, 8), lambda i: (0, 0)),
    ],
    out_specs=pl.BlockSpec((M, M, EMB, BN), lambda i: (0, 0, 0, i)),
    out_shape=jax.ShapeDtypeStruct((M, M, EMB, N), jnp.float32),
)


def kernel(traj, mat, traj_len, emb_t, emb_l, emb_u, emb_su, emb_sl, emb_tu, emb_tl):
    traj = traj.astype(jnp.int32)
    idx_u = traj[:, :, 0].reshape(ROWS)
    idx_l = traj[:, :, 1].reshape(ROWS)
    idx_t = ((traj[:, :, 2] - 1) % HOURS + 1).reshape(ROWS)

    tab_l = _pack_call(NUM_LOC, Q_L, 6400)(jnp.transpose(emb_l))
    tab_u = _pack_call(NUM_USER, Q_U, 3200)(jnp.transpose(emb_u))
    tab_t = jnp.pad(emb_t, ((0, 3), (0, 0))).reshape((HOURS + 4) // 4, 4 * EMB)

    rt, rl, ru = _gather_call()(idx_t // 4, idx_l % Q_L, idx_u % Q_U,
                                tab_t, tab_l, tab_u)
    joint = _joint_call(rt, rl, ru,
                        (idx_t % 4).reshape(ROWS, 1),
                        (idx_l // Q_L).reshape(ROWS, 1),
                        (idx_u // Q_U).reshape(ROWS, 1))

    A = emb_sl + emb_tl
    S = (emb_su - emb_sl) * (1.0 / (SU - SL))
    T = (emb_tu - emb_tl) * (1.0 / (TU - TL))
    z = jnp.zeros((EMB,), jnp.float32)
    cst = jnp.stack([A[0], A[1], S[0], S[1], T[0], T[1], z, z], axis=1)

    matT = jnp.transpose(mat, (1, 2, 3, 0))
    lent = traj_len.astype(jnp.int32).reshape(1, N)
    delta_phys = _delta_call(lent, matT, cst)
    delta = jnp.transpose(delta_phys, (3, 0, 1, 2))
    return joint.reshape(N, M, EMB), delta
